# baseline (speedup 1.0000x reference)
import functools

import jax
import jax.numpy as jnp
from jax import lax
from jax.experimental import pallas as pl
from jax.experimental.pallas import tpu as pltpu
from jax.experimental.pallas import tpu_sc as plsc

H = 128
N = 10000
E = 320000
NC = 2
NS = 16
NW = NC * NS
EPW = E // NW
CH = 80
NCHUNK = EPW // CH
SCH = 40
SNCH = EPW // SCH
RPT = 632
NP = RPT * NS
EPS = 1e-5
F32 = jnp.float32


def _ln(x, g, b):
    j = jnp.full((H, H), 1.0 / H, F32)
    mu = jnp.dot(x, j, preferred_element_type=F32)
    m2 = jnp.dot(x * x, j, preferred_element_type=F32)
    s = lax.rsqrt(m2 - mu * mu + EPS)
    return (x - mu) * (s * g) + b


def _silu(x):
    return x * (1.0 / (1.0 + jnp.exp(-x)))


def _proj_body(hn, ws, wd, ps, pd):
    x = hn[...]
    ps[...] = jnp.dot(x, ws[...], preferred_element_type=F32)
    pd[...] = jnp.dot(x, wd[...], preferred_element_type=F32)


def _edge_body(gs, gd, he, w1e, b1, g1, be1, w2, b2, eng, enb, out):
    x = he[...]
    h = gs[...] + gd[...] + b1[...] + jnp.dot(x, w1e[...], preferred_element_type=F32)
    h = _silu(_ln(h, g1[...], be1[...]))
    y = x + b2[...] + jnp.dot(h, w2[...], preferred_element_type=F32)
    out[...] = _ln(y, eng[...], enb[...])


def _node_body(hn, ap, w1a, w1b, b1, g1, be1, w2, b2, nng, nnb, out):
    x = hn[...]
    agg = ap[0] + ap[1]
    h = (b1[...] + jnp.dot(x, w1a[...], preferred_element_type=F32)
         + jnp.dot(agg, w1b[...], preferred_element_type=F32))
    h = _silu(_ln(h, g1[...], be1[...]))
    y = x + b2[...] + jnp.dot(h, w2[...], preferred_element_type=F32)
    out[...] = _ln(y, nng[...], nnb[...])


def _gather_body(ps_hbm, pd_hbm, src3_hbm, dst3_hbm, gs_hbm, gd_hbm,
                 idxs, idxd,
                 bs0, bs1, bs2, bs3, bd0, bd1, bd2, bd3,
                 gsem0, gsem1, gsem2, gsem3, wsem0, wsem1, wsem2, wsem3):
    c = lax.axis_index("c")
    s = lax.axis_index("s")
    wid = s * NC + c
    ebase = wid * EPW
    pltpu.sync_copy(src3_hbm.at[wid], idxs)
    pltpu.sync_copy(dst3_hbm.at[wid], idxd)

    bs = (bs0, bs1, bs2, bs3)
    bd = (bd0, bd1, bd2, bd3)
    gsem = (gsem0, gsem1, gsem2, gsem3)
    wsem = (wsem0, wsem1, wsem2, wsem3)

    def fire_gather(j, b):
        pltpu.async_copy(ps_hbm.at[idxs.at[j]], bs[b], gsem[b])
        pltpu.async_copy(pd_hbm.at[idxd.at[j]], bd[b], gsem[b])

    def wait_gather(b):
        pltpu.make_async_copy(ps_hbm.at[pl.ds(0, CH)], bs[b], gsem[b]).wait()
        pltpu.make_async_copy(ps_hbm.at[pl.ds(0, CH)], bd[b], gsem[b]).wait()

    def fire_write(i, b):
        off = ebase + i * CH
        pltpu.async_copy(bs[b], gs_hbm.at[pl.ds(off, CH)], wsem[b])
        pltpu.async_copy(bd[b], gd_hbm.at[pl.ds(off, CH)], wsem[b])

    def wait_write(b):
        pltpu.make_async_copy(bs[b], gs_hbm.at[pl.ds(0, CH)], wsem[b]).wait()
        pltpu.make_async_copy(bd[b], gd_hbm.at[pl.ds(0, CH)], wsem[b]).wait()

    fire_gather(0, 0)
    fire_gather(1, 1)

    def quad(k, carry):
        for b in range(4):
            i = 4 * k + b
            b2 = (b + 2) % 4
            wait_gather(b)
            fire_write(i, b)

            @pl.when(i + 2 < NCHUNK)
            def _():
                @pl.when(i >= 2)
                def _():
                    wait_write(b2)
                fire_gather(i + 2, b2)
        return carry

    lax.fori_loop(0, NCHUNK // 4, quad, 0)
    wait_gather(0)
    fire_write(NCHUNK - 1, 0)
    for b in (1, 2, 3, 0):
        wait_write(b)


def _scatter_body(he3_hbm, dst3_hbm, zz_hbm, out_hbm,
                  idxm, rows0, rows1, shared, rsem0, rsem1):
    c = lax.axis_index("c")
    s = lax.axis_index("s")
    pltpu.sync_copy(zz_hbm.at[pl.ds(s * RPT, RPT)], shared.at[pl.ds(s * RPT, RPT)])
    wid = c * NS + s
    pltpu.sync_copy(dst3_hbm.at[wid], idxm)
    plsc.subcore_barrier()

    rows = (rows0, rows1)
    rsem = (rsem0, rsem1)

    def fire_load(i, b):
        pltpu.async_copy(he3_hbm.at[wid, pl.ds(i * SCH, SCH)], rows[b], rsem[b])

    def wait_load(b):
        pltpu.make_async_copy(he3_hbm.at[0, pl.ds(0, SCH)], rows[b], rsem[b]).wait()

    fire_load(0, 0)
    fire_load(1, 1)

    def pair(k, carry):
        for b in range(2):
            i = 2 * k + b
            wait_load(b)
            pltpu.sync_copy(rows[b], shared.at[idxm.at[i]], add=True)

            @pl.when(i + 2 < SNCH)
            def _():
                fire_load(i + 2, b)
        return carry

    lax.fori_loop(0, SNCH // 2, pair, 0)
    plsc.subcore_barrier()
    pltpu.sync_copy(shared.at[pl.ds(s * RPT, RPT)], out_hbm.at[c, pl.ds(s * RPT, RPT)])


def _row(v):
    return v.reshape(1, H)


def kernel(h_node, h_edge, edge_index,
           e_W1, e_b1, e_g1, e_be1, e_W2, e_b2,
           n_W1, n_b1, n_g1, n_be1, n_W2, n_b2,
           en_g, en_b, nn_g, nn_b):
    src = edge_index[0].astype(jnp.int32)
    dst = edge_index[1].astype(jnp.int32)
    w1s, w1d, w1e = e_W1[:H], e_W1[H:2 * H], e_W1[2 * H:]
    nw1a, nw1b = n_W1[:H], n_W1[H:]

    BP = 2000
    ps, pd = pl.pallas_call(
        _proj_body,
        grid=(N // BP,),
        in_specs=[
            pl.BlockSpec((BP, H), lambda i: (i, 0)),
            pl.BlockSpec((H, H), lambda i: (0, 0)),
            pl.BlockSpec((H, H), lambda i: (0, 0)),
        ],
        out_specs=[
            pl.BlockSpec((BP, H), lambda i: (i, 0)),
            pl.BlockSpec((BP, H), lambda i: (i, 0)),
        ],
        out_shape=[jax.ShapeDtypeStruct((N, H), F32)] * 2,
    )(h_node, w1s, w1d)

    src3 = src.reshape(NW, NCHUNK, CH)
    dst3 = dst.reshape(NW, NCHUNK, CH)
    mesh = plsc.VectorSubcoreMesh(core_axis_name="c", subcore_axis_name="s")
    gs, gd = pl.kernel(
        _gather_body,
        out_type=[jax.ShapeDtypeStruct((E, H), F32)] * 2,
        mesh=mesh,
        scratch_types=(
            [pltpu.VMEM((NCHUNK, CH), jnp.int32)] * 2
            + [pltpu.VMEM((CH, H), F32)] * 8
            + [pltpu.SemaphoreType.DMA] * 8
        ),
    )(ps, pd, src3, dst3)

    BE = 1280
    wspec = pl.BlockSpec((H, H), lambda i: (0, 0))
    vspec = pl.BlockSpec((1, H), lambda i: (0, 0))
    bspec = pl.BlockSpec((BE, H), lambda i: (i, 0))
    h_edge_new = pl.pallas_call(
        _edge_body,
        grid=(E // BE,),
        in_specs=[bspec, bspec, bspec, wspec, vspec, vspec, vspec, wspec,
                  vspec, vspec, vspec],
        out_specs=bspec,
        out_shape=jax.ShapeDtypeStruct((E, H), F32),
    )(gs, gd, h_edge, w1e, _row(e_b1), _row(e_g1), _row(e_be1), e_W2,
      _row(e_b2), _row(en_g), _row(en_b))

    aggp = pl.kernel(
        _scatter_body,
        out_type=jax.ShapeDtypeStruct((NC, NP, H), F32),
        mesh=plsc.VectorSubcoreMesh(core_axis_name="c", subcore_axis_name="s"),
        scratch_types=(
            [pltpu.VMEM((SNCH, SCH), jnp.int32)]
            + [pltpu.VMEM((SCH, H), F32)] * 2
            + [pltpu.VMEM_SHARED((NP, H), F32)]
            + [pltpu.SemaphoreType.DMA] * 2
        ),
    )(h_edge_new.reshape(NW, EPW, H), dst.reshape(NW, SNCH, SCH),
      jnp.zeros((NP, H), F32))

    BN = 2000
    wspecn = pl.BlockSpec((H, H), lambda i: (0, 0))
    vspecn = pl.BlockSpec((1, H), lambda i: (0, 0))
    h_node_new = pl.pallas_call(
        _node_body,
        grid=(N // BN,),
        in_specs=[
            pl.BlockSpec((BN, H), lambda i: (i, 0)),
            pl.BlockSpec((NC, BN, H), lambda i: (0, i, 0)),
            wspecn, wspecn, vspecn, vspecn, vspecn, wspecn, vspecn,
            vspecn, vspecn,
        ],
        out_specs=pl.BlockSpec((BN, H), lambda i: (i, 0)),
        out_shape=jax.ShapeDtypeStruct((N, H), F32),
    )(h_node, aggp, nw1a, nw1b, _row(n_b1), _row(n_g1), _row(n_be1), n_W2,
      _row(n_b2), _row(nn_g), _row(nn_b))

    return h_node_new, h_edge_new


# speedup vs baseline: 1.1071x; 1.1071x over previous
import functools

import jax
import jax.numpy as jnp
from jax import lax
from jax.experimental import pallas as pl
from jax.experimental.pallas import tpu as pltpu
from jax.experimental.pallas import tpu_sc as plsc

H = 128
N = 10000
E = 320000
NC = 2
NS = 16
NW = NC * NS
EPW = E // NW
CH = 80
NCHUNK = EPW // CH
SCH = 40
SNCH = EPW // SCH
RPT = 632
NP = RPT * NS
EPS = 1e-5
F32 = jnp.float32
BF16 = jnp.bfloat16


def _ln(x, g, b):
    j = jnp.full((H, H), 1.0 / H, F32)
    mu = jnp.dot(x, j, preferred_element_type=F32)
    m2 = jnp.dot(x * x, j, preferred_element_type=F32)
    s = lax.rsqrt(m2 - mu * mu + EPS)
    return (x - mu) * (s * g) + b


def _silu(x):
    return x * (1.0 / (1.0 + jnp.exp(-x)))


def _proj_body(hn, ws, wd, ps, pd):
    x = hn[...]
    ps[...] = jnp.dot(x, ws[...], preferred_element_type=F32)
    pd[...] = jnp.dot(x, wd[...], preferred_element_type=F32)


def _edge_body(gs, gd, he, w1e, b1, g1, be1, w2, b2, eng, enb, out):
    x = he[...]
    h = (gs[...] + gd[...] + b1[...]
         + jnp.dot(x, w1e[...], preferred_element_type=F32))
    h = _silu(_ln(h, g1[...], be1[...]))
    y = x + b2[...] + jnp.dot(h, w2[...], preferred_element_type=F32)
    out[...] = _ln(y, eng[...], enb[...])


def _node_body(hn, ap, w1a, w1b, b1, g1, be1, w2, b2, nng, nnb, out):
    x = hn[...]
    agg = ap[0] + ap[1]
    h = (b1[...] + jnp.dot(x, w1a[...], preferred_element_type=F32)
         + jnp.dot(agg, w1b[...], preferred_element_type=F32))
    h = _silu(_ln(h, g1[...], be1[...]))
    y = x + b2[...] + jnp.dot(h, w2[...], preferred_element_type=F32)
    out[...] = _ln(y, nng[...], nnb[...])


def _gather_body(ps_hbm, pd_hbm, src3_hbm, dst3_hbm, gs_hbm, gd_hbm,
                 idxs, idxd,
                 bs0, bs1, bs2, bs3, bd0, bd1, bd2, bd3,
                 gsem0, gsem1, gsem2, gsem3, wsem0, wsem1, wsem2, wsem3):
    c = lax.axis_index("c")
    s = lax.axis_index("s")
    wid = s * NC + c
    ebase = wid * EPW
    pltpu.sync_copy(src3_hbm.at[wid], idxs)
    pltpu.sync_copy(dst3_hbm.at[wid], idxd)

    bs = (bs0, bs1, bs2, bs3)
    bd = (bd0, bd1, bd2, bd3)
    gsem = (gsem0, gsem1, gsem2, gsem3)
    wsem = (wsem0, wsem1, wsem2, wsem3)

    def fire_gather(j, b):
        pltpu.async_copy(ps_hbm.at[idxs.at[j]], bs[b], gsem[b])
        pltpu.async_copy(pd_hbm.at[idxd.at[j]], bd[b], gsem[b])

    def wait_gather(b):
        pltpu.make_async_copy(ps_hbm.at[pl.ds(0, CH)], bs[b], gsem[b]).wait()
        pltpu.make_async_copy(ps_hbm.at[pl.ds(0, CH)], bd[b], gsem[b]).wait()

    def fire_write(i, b):
        off = ebase + i * CH
        pltpu.async_copy(bs[b], gs_hbm.at[pl.ds(off, CH)], wsem[b])
        pltpu.async_copy(bd[b], gd_hbm.at[pl.ds(off, CH)], wsem[b])

    def wait_write(b):
        pltpu.make_async_copy(bs[b], gs_hbm.at[pl.ds(0, CH)], wsem[b]).wait()
        pltpu.make_async_copy(bd[b], gd_hbm.at[pl.ds(0, CH)], wsem[b]).wait()

    fire_gather(0, 0)
    fire_gather(1, 1)

    def quad(k, carry):
        for b in range(4):
            i = 4 * k + b
            b2 = (b + 2) % 4
            wait_gather(b)
            fire_write(i, b)

            @pl.when(i + 2 < NCHUNK)
            def _():
                @pl.when(i >= 2)
                def _():
                    wait_write(b2)
                fire_gather(i + 2, b2)
        return carry

    lax.fori_loop(0, NCHUNK // 4, quad, 0)
    wait_gather(0)
    fire_write(NCHUNK - 1, 0)
    for b in (1, 2, 3, 0):
        wait_write(b)


def _scatter_body(he3_hbm, dst3_hbm, zz_hbm, out_hbm,
                  idxm, rows0, rows1, shared, rsem0, rsem1):
    c = lax.axis_index("c")
    s = lax.axis_index("s")
    pltpu.sync_copy(zz_hbm.at[pl.ds(s * RPT, RPT)], shared.at[pl.ds(s * RPT, RPT)])
    wid = c * NS + s
    pltpu.sync_copy(dst3_hbm.at[wid], idxm)
    plsc.subcore_barrier()

    rows = (rows0, rows1)
    rsem = (rsem0, rsem1)

    def fire_load(i, b):
        pltpu.async_copy(he3_hbm.at[wid, pl.ds(i * SCH, SCH)], rows[b], rsem[b])

    def wait_load(b):
        pltpu.make_async_copy(he3_hbm.at[0, pl.ds(0, SCH)], rows[b], rsem[b]).wait()

    fire_load(0, 0)
    fire_load(1, 1)

    def pair(k, carry):
        for b in range(2):
            i = 2 * k + b
            wait_load(b)
            pltpu.sync_copy(rows[b], shared.at[idxm.at[i]], add=True)

            @pl.when(i + 2 < SNCH)
            def _():
                fire_load(i + 2, b)
        return carry

    lax.fori_loop(0, SNCH // 2, pair, 0)
    plsc.subcore_barrier()
    pltpu.sync_copy(shared.at[pl.ds(s * RPT, RPT)], out_hbm.at[c, pl.ds(s * RPT, RPT)])


def _row(v):
    return v.reshape(1, H)


def kernel(h_node, h_edge, edge_index,
           e_W1, e_b1, e_g1, e_be1, e_W2, e_b2,
           n_W1, n_b1, n_g1, n_be1, n_W2, n_b2,
           en_g, en_b, nn_g, nn_b):
    src = edge_index[0].astype(jnp.int32)
    dst = edge_index[1].astype(jnp.int32)
    w1s, w1d, w1e = e_W1[:H], e_W1[H:2 * H], e_W1[2 * H:]
    nw1a, nw1b = n_W1[:H], n_W1[H:]

    BP = 2000
    ps, pd = pl.pallas_call(
        _proj_body,
        grid=(N // BP,),
        in_specs=[
            pl.BlockSpec((BP, H), lambda i: (i, 0)),
            pl.BlockSpec((H, H), lambda i: (0, 0)),
            pl.BlockSpec((H, H), lambda i: (0, 0)),
        ],
        out_specs=[
            pl.BlockSpec((BP, H), lambda i: (i, 0)),
            pl.BlockSpec((BP, H), lambda i: (i, 0)),
        ],
        out_shape=[jax.ShapeDtypeStruct((N, H), F32)] * 2,
    )(h_node, w1s, w1d)

    src3 = src.reshape(NW, NCHUNK, CH)
    dst3 = dst.reshape(NW, NCHUNK, CH)
    mesh = plsc.VectorSubcoreMesh(core_axis_name="c", subcore_axis_name="s")
    gs, gd = pl.kernel(
        _gather_body,
        out_type=[jax.ShapeDtypeStruct((E, H), F32)] * 2,
        mesh=mesh,
        scratch_types=(
            [pltpu.VMEM((NCHUNK, CH), jnp.int32)] * 2
            + [pltpu.VMEM((CH, H), F32)] * 8
            + [pltpu.SemaphoreType.DMA] * 8
        ),
    )(ps, pd, src3, dst3)

    BE = 2560
    wspec = pl.BlockSpec((H, H), lambda i: (0, 0))
    vspec = pl.BlockSpec((1, H), lambda i: (0, 0))
    bspec = pl.BlockSpec((BE, H), lambda i: (i, 0))
    h_edge_new = pl.pallas_call(
        _edge_body,
        grid=(E // BE,),
        in_specs=[bspec, bspec, bspec, wspec, vspec, vspec, vspec, wspec,
                  vspec, vspec, vspec],
        out_specs=bspec,
        out_shape=jax.ShapeDtypeStruct((E, H), F32),
    )(gs, gd, h_edge, w1e, _row(e_b1), _row(e_g1), _row(e_be1), e_W2,
      _row(e_b2), _row(en_g), _row(en_b))

    aggp = pl.kernel(
        _scatter_body,
        out_type=jax.ShapeDtypeStruct((NC, NP, H), F32),
        mesh=plsc.VectorSubcoreMesh(core_axis_name="c", subcore_axis_name="s"),
        scratch_types=(
            [pltpu.VMEM((SNCH, SCH), jnp.int32)]
            + [pltpu.VMEM((SCH, H), F32)] * 2
            + [pltpu.VMEM_SHARED((NP, H), F32)]
            + [pltpu.SemaphoreType.DMA] * 2
        ),
    )(h_edge_new.reshape(NW, EPW, H), dst.reshape(NW, SNCH, SCH),
      jnp.zeros((NP, H), F32))

    BN = 2000
    wspecn = pl.BlockSpec((H, H), lambda i: (0, 0))
    vspecn = pl.BlockSpec((1, H), lambda i: (0, 0))
    h_node_new = pl.pallas_call(
        _node_body,
        grid=(N // BN,),
        in_specs=[
            pl.BlockSpec((BN, H), lambda i: (i, 0)),
            pl.BlockSpec((NC, BN, H), lambda i: (0, i, 0)),
            wspecn, wspecn, vspecn, vspecn, vspecn, wspecn, vspecn,
            vspecn, vspecn,
        ],
        out_specs=pl.BlockSpec((BN, H), lambda i: (i, 0)),
        out_shape=jax.ShapeDtypeStruct((N, H), F32),
    )(h_node, aggp, nw1a, nw1b, _row(n_b1), _row(n_g1), _row(n_be1), n_W2,
      _row(n_b2), _row(nn_g), _row(nn_b))

    return h_node_new, h_edge_new


# speedup vs baseline: 1.1534x; 1.0418x over previous
import functools

import jax
import jax.numpy as jnp
from jax import lax
from jax.experimental import pallas as pl
from jax.experimental.pallas import tpu as pltpu
from jax.experimental.pallas import tpu_sc as plsc

H = 128
N = 10000
E = 320000
NC = 2
NS = 16
NW = NC * NS
EPW = E // NW
CH = 80
NCHUNK = EPW // CH
SCH = 40
SNCH = EPW // SCH
RPT = 632
NP = RPT * NS
EPS = 1e-5
F32 = jnp.float32
BF16 = jnp.bfloat16


def _ln(x, g, b):
    j = jnp.full((H, H), 1.0 / H, F32)
    mu = jnp.dot(x, j, preferred_element_type=F32)
    m2 = jnp.dot(x * x, j, preferred_element_type=F32)
    s = lax.rsqrt(m2 - mu * mu + EPS)
    return (x - mu) * (s * g) + b


def _silu(x):
    return x * (1.0 / (1.0 + jnp.exp(-x)))


def _proj_body(hn, ws, wd, ps, pd):
    x = hn[...]
    ps[...] = jnp.dot(x, ws[...], preferred_element_type=F32)
    pd[...] = jnp.dot(x, wd[...], preferred_element_type=F32)


def _edge_body(gs, gd, he, w1e, b1, g1, be1, w2, b2, eng, enb, out):
    x = he[...]
    h = (gs[...] + gd[...] + b1[...]
         + jnp.dot(x, w1e[...], preferred_element_type=F32))
    h = _silu(_ln(h, g1[...], be1[...]))
    y = x + b2[...] + jnp.dot(h, w2[...], preferred_element_type=F32)
    out[...] = _ln(y, eng[...], enb[...])


def _node_body(hn, ap, w1a, w1b, b1, g1, be1, w2, b2, nng, nnb, out):
    x = hn[...]
    agg = ap[0] + ap[1]
    h = (b1[...] + jnp.dot(x, w1a[...], preferred_element_type=F32)
         + jnp.dot(agg, w1b[...], preferred_element_type=F32))
    h = _silu(_ln(h, g1[...], be1[...]))
    y = x + b2[...] + jnp.dot(h, w2[...], preferred_element_type=F32)
    out[...] = _ln(y, nng[...], nnb[...])


def _gather_body(ps_hbm, pd_hbm, src3_hbm, dst3_hbm, gs_hbm, gd_hbm,
                 idxs, idxd,
                 bs0, bs1, bs2, bs3, bd0, bd1, bd2, bd3,
                 gsem0, gsem1, gsem2, gsem3, wsem0, wsem1, wsem2, wsem3):
    c = lax.axis_index("c")
    s = lax.axis_index("s")
    wid = s * NC + c
    ebase = wid * EPW
    pltpu.sync_copy(src3_hbm.at[wid], idxs)
    pltpu.sync_copy(dst3_hbm.at[wid], idxd)

    bs = (bs0, bs1, bs2, bs3)
    bd = (bd0, bd1, bd2, bd3)
    gsem = (gsem0, gsem1, gsem2, gsem3)
    wsem = (wsem0, wsem1, wsem2, wsem3)

    def fire_gather(j, b):
        pltpu.async_copy(ps_hbm.at[idxs.at[j]], bs[b], gsem[b])
        pltpu.async_copy(pd_hbm.at[idxd.at[j]], bd[b], gsem[b])

    def wait_gather(b):
        pltpu.make_async_copy(ps_hbm.at[pl.ds(0, CH)], bs[b], gsem[b]).wait()
        pltpu.make_async_copy(ps_hbm.at[pl.ds(0, CH)], bd[b], gsem[b]).wait()

    def fire_write(i, b):
        off = ebase + i * CH
        pltpu.async_copy(bs[b], gs_hbm.at[pl.ds(off, CH)], wsem[b])
        pltpu.async_copy(bd[b], gd_hbm.at[pl.ds(off, CH)], wsem[b])

    def wait_write(b):
        pltpu.make_async_copy(bs[b], gs_hbm.at[pl.ds(0, CH)], wsem[b]).wait()
        pltpu.make_async_copy(bd[b], gd_hbm.at[pl.ds(0, CH)], wsem[b]).wait()

    fire_gather(0, 0)
    fire_gather(1, 1)

    def quad(k, carry):
        for b in range(4):
            i = 4 * k + b
            b2 = (b + 2) % 4
            wait_gather(b)
            fire_write(i, b)

            @pl.when(i + 2 < NCHUNK)
            def _():
                @pl.when(i >= 2)
                def _():
                    wait_write(b2)
                fire_gather(i + 2, b2)
        return carry

    lax.fori_loop(0, NCHUNK // 4, quad, 0)
    wait_gather(0)
    fire_write(NCHUNK - 1, 0)
    for b in (1, 2, 3, 0):
        wait_write(b)


def _scatter_body(he3_hbm, dst3_hbm, zz_hbm, out_hbm,
                  idxm, rows0, rows1, shared, rsem0, rsem1):
    c = lax.axis_index("c")
    s = lax.axis_index("s")
    pltpu.sync_copy(zz_hbm.at[pl.ds(s * RPT, RPT)], shared.at[pl.ds(s * RPT, RPT)])
    wid = c * NS + s
    pltpu.sync_copy(dst3_hbm.at[wid], idxm)
    plsc.subcore_barrier()

    rows = (rows0, rows1)
    rsem = (rsem0, rsem1)

    def fire_load(i, b):
        pltpu.async_copy(he3_hbm.at[wid, pl.ds(i * SCH, SCH)], rows[b], rsem[b])

    def wait_load(b):
        pltpu.make_async_copy(he3_hbm.at[0, pl.ds(0, SCH)], rows[b], rsem[b]).wait()

    fire_load(0, 0)
    fire_load(1, 1)

    def pair(k, carry):
        for b in range(2):
            i = 2 * k + b
            wait_load(b)
            pltpu.sync_copy(rows[b], shared.at[idxm.at[i]], add=True)

            @pl.when(i + 2 < SNCH)
            def _():
                fire_load(i + 2, b)
        return carry

    lax.fori_loop(0, SNCH // 2, pair, 0)
    plsc.subcore_barrier()
    pltpu.sync_copy(shared.at[pl.ds(s * RPT, RPT)], out_hbm.at[c, pl.ds(s * RPT, RPT)])


def _row(v):
    return v.reshape(1, H)


def kernel(h_node, h_edge, edge_index,
           e_W1, e_b1, e_g1, e_be1, e_W2, e_b2,
           n_W1, n_b1, n_g1, n_be1, n_W2, n_b2,
           en_g, en_b, nn_g, nn_b):
    src = edge_index[0].astype(jnp.int32)
    dst = edge_index[1].astype(jnp.int32)
    w1s, w1d, w1e = e_W1[:H], e_W1[H:2 * H], e_W1[2 * H:]
    nw1a, nw1b = n_W1[:H], n_W1[H:]

    BP = 2000
    ps, pd = pl.pallas_call(
        _proj_body,
        grid=(N // BP,),
        in_specs=[
            pl.BlockSpec((BP, H), lambda i: (i, 0)),
            pl.BlockSpec((H, H), lambda i: (0, 0)),
            pl.BlockSpec((H, H), lambda i: (0, 0)),
        ],
        out_specs=[
            pl.BlockSpec((BP, H), lambda i: (i, 0)),
            pl.BlockSpec((BP, H), lambda i: (i, 0)),
        ],
        out_shape=[jax.ShapeDtypeStruct((N, H), F32)] * 2,
    )(h_node, w1s, w1d)

    src3 = src.reshape(NW, NCHUNK, CH)
    dst3 = dst.reshape(NW, NCHUNK, CH)
    mesh = plsc.VectorSubcoreMesh(core_axis_name="c", subcore_axis_name="s")
    gs, gd = pl.kernel(
        _gather_body,
        out_type=[jax.ShapeDtypeStruct((E, H), F32)] * 2,
        mesh=mesh,
        scratch_types=(
            [pltpu.VMEM((NCHUNK, CH), jnp.int32)] * 2
            + [pltpu.VMEM((CH, H), F32)] * 8
            + [pltpu.SemaphoreType.DMA] * 8
        ),
    )(ps, pd, src3, dst3)

    BE = 4000
    wspec = pl.BlockSpec((H, H), lambda i: (0, 0))
    vspec = pl.BlockSpec((1, H), lambda i: (0, 0))
    bspec = pl.BlockSpec((BE, H), lambda i: (i, 0))
    h_edge_new = pl.pallas_call(
        _edge_body,
        grid=(E // BE,),
        in_specs=[bspec, bspec, bspec, wspec, vspec, vspec, vspec, wspec,
                  vspec, vspec, vspec],
        out_specs=bspec,
        out_shape=jax.ShapeDtypeStruct((E, H), F32),
    )(gs, gd, h_edge, w1e, _row(e_b1), _row(e_g1), _row(e_be1), e_W2,
      _row(e_b2), _row(en_g), _row(en_b))

    aggp = pl.kernel(
        _scatter_body,
        out_type=jax.ShapeDtypeStruct((NC, NP, H), F32),
        mesh=plsc.VectorSubcoreMesh(core_axis_name="c", subcore_axis_name="s"),
        scratch_types=(
            [pltpu.VMEM((SNCH, SCH), jnp.int32)]
            + [pltpu.VMEM((SCH, H), F32)] * 2
            + [pltpu.VMEM_SHARED((NP, H), F32)]
            + [pltpu.SemaphoreType.DMA] * 2
        ),
    )(h_edge_new.reshape(NW, EPW, H), dst.reshape(NW, SNCH, SCH),
      jnp.zeros((NP, H), F32))

    BN = 2000
    wspecn = pl.BlockSpec((H, H), lambda i: (0, 0))
    vspecn = pl.BlockSpec((1, H), lambda i: (0, 0))
    h_node_new = pl.pallas_call(
        _node_body,
        grid=(N // BN,),
        in_specs=[
            pl.BlockSpec((BN, H), lambda i: (i, 0)),
            pl.BlockSpec((NC, BN, H), lambda i: (0, i, 0)),
            wspecn, wspecn, vspecn, vspecn, vspecn, wspecn, vspecn,
            vspecn, vspecn,
        ],
        out_specs=pl.BlockSpec((BN, H), lambda i: (i, 0)),
        out_shape=jax.ShapeDtypeStruct((N, H), F32),
    )(h_node, aggp, nw1a, nw1b, _row(n_b1), _row(n_g1), _row(n_be1), n_W2,
      _row(n_b2), _row(nn_g), _row(nn_b))

    return h_node_new, h_edge_new


# speedup vs baseline: 1.1987x; 1.0393x over previous
import functools

import jax
import jax.numpy as jnp
from jax import lax
from jax.experimental import pallas as pl
from jax.experimental.pallas import tpu as pltpu
from jax.experimental.pallas import tpu_sc as plsc

H = 128
N = 10000
E = 320000
NC = 2
NS = 16
NW = NC * NS
EPW = E // NW
CH = 80
NCHUNK = EPW // CH
SCH = 40
SNCH = EPW // SCH
RPT = 632
NP = RPT * NS
EPS = 1e-5
F32 = jnp.float32
BF16 = jnp.bfloat16


def _ln(x, g, b):
    j = jnp.full((H, H), 1.0 / H, F32)
    mu = jnp.dot(x, j, preferred_element_type=F32)
    m2 = jnp.dot(x * x, j, preferred_element_type=F32)
    s = lax.rsqrt(m2 - mu * mu + EPS)
    return (x - mu) * (s * g) + b


def _silu(x):
    return x * (1.0 / (1.0 + jnp.exp(-x)))


def _proj_body(hn, ws, wd, ps, pd):
    x = hn[...]
    ps[...] = jnp.dot(x, ws[...], preferred_element_type=F32)
    pd[...] = jnp.dot(x, wd[...], preferred_element_type=F32)


def _edge_body(gs, gd, he, w1e, b1, g1, be1, w2, b2, eng, enb, out):
    x = he[...]
    h = (gs[...] + gd[...] + b1[...]
         + jnp.dot(x, w1e[...], preferred_element_type=F32))
    h = _silu(_ln(h, g1[...], be1[...]))
    y = x + b2[...] + jnp.dot(h, w2[...], preferred_element_type=F32)
    out[...] = _ln(y, eng[...], enb[...])


def _node_body(hn, ap, w1a, w1b, b1, g1, be1, w2, b2, nng, nnb, out):
    x = hn[...]
    agg = ap[0] + ap[1]
    h = (b1[...] + jnp.dot(x, w1a[...], preferred_element_type=F32)
         + jnp.dot(agg, w1b[...], preferred_element_type=F32))
    h = _silu(_ln(h, g1[...], be1[...]))
    y = x + b2[...] + jnp.dot(h, w2[...], preferred_element_type=F32)
    out[...] = _ln(y, nng[...], nnb[...])


def _gather_body(ps_hbm, pd_hbm, src3_hbm, dst3_hbm, gs_hbm, gd_hbm,
                 idxs, idxd,
                 bs0, bs1, bs2, bs3, bd0, bd1, bd2, bd3,
                 gsem0, gsem1, gsem2, gsem3, wsem0, wsem1, wsem2, wsem3):
    c = lax.axis_index("c")
    s = lax.axis_index("s")
    wid = s * NC + c
    ebase = wid * EPW
    pltpu.sync_copy(src3_hbm.at[wid], idxs)
    pltpu.sync_copy(dst3_hbm.at[wid], idxd)

    bs = (bs0, bs1, bs2, bs3)
    bd = (bd0, bd1, bd2, bd3)
    gsem = (gsem0, gsem1, gsem2, gsem3)
    wsem = (wsem0, wsem1, wsem2, wsem3)

    def fire_gather(j, b):
        pltpu.async_copy(ps_hbm.at[idxs.at[j]], bs[b], gsem[b])
        pltpu.async_copy(pd_hbm.at[idxd.at[j]], bd[b], gsem[b])

    def wait_gather(b):
        pltpu.make_async_copy(ps_hbm.at[pl.ds(0, CH)], bs[b], gsem[b]).wait()
        pltpu.make_async_copy(ps_hbm.at[pl.ds(0, CH)], bd[b], gsem[b]).wait()

    def fire_write(i, b):
        off = ebase + i * CH
        pltpu.async_copy(bs[b], gs_hbm.at[pl.ds(off, CH)], wsem[b])
        pltpu.async_copy(bd[b], gd_hbm.at[pl.ds(off, CH)], wsem[b])

    def wait_write(b):
        pltpu.make_async_copy(bs[b], gs_hbm.at[pl.ds(0, CH)], wsem[b]).wait()
        pltpu.make_async_copy(bd[b], gd_hbm.at[pl.ds(0, CH)], wsem[b]).wait()

    fire_gather(0, 0)
    fire_gather(1, 1)

    def quad(k, carry):
        for b in range(4):
            i = 4 * k + b
            b2 = (b + 2) % 4
            wait_gather(b)
            fire_write(i, b)

            @pl.when(i + 2 < NCHUNK)
            def _():
                @pl.when(i >= 2)
                def _():
                    wait_write(b2)
                fire_gather(i + 2, b2)
        return carry

    lax.fori_loop(0, NCHUNK // 4, quad, 0)
    wait_gather(0)
    fire_write(NCHUNK - 1, 0)
    for b in (1, 2, 3, 0):
        wait_write(b)


def _scatter_body(he3_hbm, dst3_hbm, zz_hbm, out_hbm,
                  idxm, rows0, rows1, shared, rsem0, rsem1):
    c = lax.axis_index("c")
    s = lax.axis_index("s")
    pltpu.sync_copy(zz_hbm.at[pl.ds(s * RPT, RPT)], shared.at[pl.ds(s * RPT, RPT)])
    wid = c * NS + s
    pltpu.sync_copy(dst3_hbm.at[wid], idxm)
    plsc.subcore_barrier()

    rows = (rows0, rows1)
    rsem = (rsem0, rsem1)

    def fire_load(i, b):
        pltpu.async_copy(he3_hbm.at[wid, pl.ds(i * SCH, SCH)], rows[b], rsem[b])

    def wait_load(b):
        pltpu.make_async_copy(he3_hbm.at[0, pl.ds(0, SCH)], rows[b], rsem[b]).wait()

    fire_load(0, 0)
    fire_load(1, 1)

    def pair(k, carry):
        for b in range(2):
            i = 2 * k + b
            wait_load(b)
            pltpu.sync_copy(rows[b], shared.at[idxm.at[i]], add=True)

            @pl.when(i + 2 < SNCH)
            def _():
                fire_load(i + 2, b)
        return carry

    lax.fori_loop(0, SNCH // 2, pair, 0)
    plsc.subcore_barrier()
    pltpu.sync_copy(shared.at[pl.ds(s * RPT, RPT)], out_hbm.at[c, pl.ds(s * RPT, RPT)])


def _row(v):
    return v.reshape(1, H)


def kernel(h_node, h_edge, edge_index,
           e_W1, e_b1, e_g1, e_be1, e_W2, e_b2,
           n_W1, n_b1, n_g1, n_be1, n_W2, n_b2,
           en_g, en_b, nn_g, nn_b):
    src = edge_index[0].astype(jnp.int32)
    dst = edge_index[1].astype(jnp.int32)
    w1s, w1d, w1e = e_W1[:H], e_W1[H:2 * H], e_W1[2 * H:]
    nw1a, nw1b = n_W1[:H], n_W1[H:]

    BP = 2000
    ps, pd = pl.pallas_call(
        _proj_body,
        grid=(N // BP,),
        in_specs=[
            pl.BlockSpec((BP, H), lambda i: (i, 0)),
            pl.BlockSpec((H, H), lambda i: (0, 0)),
            pl.BlockSpec((H, H), lambda i: (0, 0)),
        ],
        out_specs=[
            pl.BlockSpec((BP, H), lambda i: (i, 0)),
            pl.BlockSpec((BP, H), lambda i: (i, 0)),
        ],
        out_shape=[jax.ShapeDtypeStruct((N, H), F32)] * 2,
    )(h_node, w1s, w1d)

    src3 = src.reshape(NW, NCHUNK, CH)
    dst3 = dst.reshape(NW, NCHUNK, CH)
    mesh = plsc.VectorSubcoreMesh(core_axis_name="c", subcore_axis_name="s")
    gs, gd = pl.kernel(
        _gather_body,
        out_type=[jax.ShapeDtypeStruct((E, H), F32)] * 2,
        mesh=mesh,
        scratch_types=(
            [pltpu.VMEM((NCHUNK, CH), jnp.int32)] * 2
            + [pltpu.VMEM((CH, H), F32)] * 8
            + [pltpu.SemaphoreType.DMA] * 8
        ),
    )(ps, pd, src3, dst3)

    BE = 8000
    wspec = pl.BlockSpec((H, H), lambda i: (0, 0))
    vspec = pl.BlockSpec((1, H), lambda i: (0, 0))
    bspec = pl.BlockSpec((BE, H), lambda i: (i, 0))
    h_edge_new = pl.pallas_call(
        _edge_body,
        grid=(E // BE,),
        in_specs=[bspec, bspec, bspec, wspec, vspec, vspec, vspec, wspec,
                  vspec, vspec, vspec],
        out_specs=bspec,
        out_shape=jax.ShapeDtypeStruct((E, H), F32),
    )(gs, gd, h_edge, w1e, _row(e_b1), _row(e_g1), _row(e_be1), e_W2,
      _row(e_b2), _row(en_g), _row(en_b))

    aggp = pl.kernel(
        _scatter_body,
        out_type=jax.ShapeDtypeStruct((NC, NP, H), F32),
        mesh=plsc.VectorSubcoreMesh(core_axis_name="c", subcore_axis_name="s"),
        scratch_types=(
            [pltpu.VMEM((SNCH, SCH), jnp.int32)]
            + [pltpu.VMEM((SCH, H), F32)] * 2
            + [pltpu.VMEM_SHARED((NP, H), F32)]
            + [pltpu.SemaphoreType.DMA] * 2
        ),
    )(h_edge_new.reshape(NW, EPW, H), dst.reshape(NW, SNCH, SCH),
      jnp.zeros((NP, H), F32))

    BN = 2000
    wspecn = pl.BlockSpec((H, H), lambda i: (0, 0))
    vspecn = pl.BlockSpec((1, H), lambda i: (0, 0))
    h_node_new = pl.pallas_call(
        _node_body,
        grid=(N // BN,),
        in_specs=[
            pl.BlockSpec((BN, H), lambda i: (i, 0)),
            pl.BlockSpec((NC, BN, H), lambda i: (0, i, 0)),
            wspecn, wspecn, vspecn, vspecn, vspecn, wspecn, vspecn,
            vspecn, vspecn,
        ],
        out_specs=pl.BlockSpec((BN, H), lambda i: (i, 0)),
        out_shape=jax.ShapeDtypeStruct((N, H), F32),
    )(h_node, aggp, nw1a, nw1b, _row(n_b1), _row(n_g1), _row(n_be1), n_W2,
      _row(n_b2), _row(nn_g), _row(nn_b))

    return h_node_new, h_edge_new


# speedup vs baseline: 1.2106x; 1.0100x over previous
import functools

import jax
import jax.numpy as jnp
from jax import lax
from jax.experimental import pallas as pl
from jax.experimental.pallas import tpu as pltpu
from jax.experimental.pallas import tpu_sc as plsc

H = 128
N = 10000
E = 320000
NC = 2
NS = 16
NW = NC * NS
EPW = E // NW
CH = 80
NCHUNK = EPW // CH
SCH = 40
SNCH = EPW // SCH
RPT = 632
NP = RPT * NS
EPS = 1e-5
F32 = jnp.float32
BF16 = jnp.bfloat16


def _ln(x, g, b):
    j = jnp.full((H, H), 1.0 / H, F32)
    mu = jnp.dot(x, j, preferred_element_type=F32)
    m2 = jnp.dot(x * x, j, preferred_element_type=F32)
    s = lax.rsqrt(m2 - mu * mu + EPS)
    return (x - mu) * (s * g) + b


def _silu(x):
    return x * (1.0 / (1.0 + jnp.exp(-x)))


def _proj_body(hn, ws, wd, ps, pd):
    x = hn[...]
    ps[...] = jnp.dot(x, ws[...], preferred_element_type=F32)
    pd[...] = jnp.dot(x, wd[...], preferred_element_type=F32)


def _edge_body(gs, gd, he, w1e, b1, g1, be1, w2, b2, eng, enb, out):
    x = he[...]
    h = (gs[...] + gd[...] + b1[...]
         + jnp.dot(x, w1e[...], preferred_element_type=F32))
    h = _silu(_ln(h, g1[...], be1[...]))
    y = x + b2[...] + jnp.dot(h, w2[...], preferred_element_type=F32)
    out[...] = _ln(y, eng[...], enb[...])


def _node_body(hn, ap, w1a, w1b, b1, g1, be1, w2, b2, nng, nnb, out):
    x = hn[...]
    agg = ap[0] + ap[1]
    h = (b1[...] + jnp.dot(x, w1a[...], preferred_element_type=F32)
         + jnp.dot(agg, w1b[...], preferred_element_type=F32))
    h = _silu(_ln(h, g1[...], be1[...]))
    y = x + b2[...] + jnp.dot(h, w2[...], preferred_element_type=F32)
    out[...] = _ln(y, nng[...], nnb[...])


def _gather_body(ps_hbm, pd_hbm, src3_hbm, dst3_hbm, gs_hbm, gd_hbm,
                 idxs, idxd,
                 bs0, bs1, bs2, bs3, bd0, bd1, bd2, bd3,
                 gsem0, gsem1, gsem2, gsem3, wsem0, wsem1, wsem2, wsem3):
    c = lax.axis_index("c")
    s = lax.axis_index("s")
    wid = s * NC + c
    ebase = wid * EPW
    pltpu.sync_copy(src3_hbm.at[wid], idxs)
    pltpu.sync_copy(dst3_hbm.at[wid], idxd)

    bs = (bs0, bs1, bs2, bs3)
    bd = (bd0, bd1, bd2, bd3)
    gsem = (gsem0, gsem1, gsem2, gsem3)
    wsem = (wsem0, wsem1, wsem2, wsem3)

    def fire_gather(j, b):
        pltpu.async_copy(ps_hbm.at[idxs.at[j]], bs[b], gsem[b])
        pltpu.async_copy(pd_hbm.at[idxd.at[j]], bd[b], gsem[b])

    def wait_gather(b):
        pltpu.make_async_copy(ps_hbm.at[pl.ds(0, CH)], bs[b], gsem[b]).wait()
        pltpu.make_async_copy(ps_hbm.at[pl.ds(0, CH)], bd[b], gsem[b]).wait()

    def fire_write(i, b):
        off = ebase + i * CH
        pltpu.async_copy(bs[b], gs_hbm.at[pl.ds(off, CH)], wsem[b])
        pltpu.async_copy(bd[b], gd_hbm.at[pl.ds(off, CH)], wsem[b])

    def wait_write(b):
        pltpu.make_async_copy(bs[b], gs_hbm.at[pl.ds(0, CH)], wsem[b]).wait()
        pltpu.make_async_copy(bd[b], gd_hbm.at[pl.ds(0, CH)], wsem[b]).wait()

    fire_gather(0, 0)
    fire_gather(1, 1)

    def quad(k, carry):
        for b in range(4):
            i = 4 * k + b
            b2 = (b + 2) % 4
            wait_gather(b)
            fire_write(i, b)

            @pl.when(i + 2 < NCHUNK)
            def _():
                @pl.when(i >= 2)
                def _():
                    wait_write(b2)
                fire_gather(i + 2, b2)
        return carry

    lax.fori_loop(0, NCHUNK // 4, quad, 0)
    wait_gather(0)
    fire_write(NCHUNK - 1, 0)
    for b in (1, 2, 3, 0):
        wait_write(b)


def _scatter_body(he3_hbm, dst3_hbm, zz_hbm, out_hbm,
                  idxm, rows0, rows1, shared, rsem0, rsem1):
    c = lax.axis_index("c")
    s = lax.axis_index("s")
    pltpu.sync_copy(zz_hbm.at[pl.ds(s * RPT, RPT)], shared.at[pl.ds(s * RPT, RPT)])
    wid = c * NS + s
    pltpu.sync_copy(dst3_hbm.at[wid], idxm)
    plsc.subcore_barrier()

    rows = (rows0, rows1)
    rsem = (rsem0, rsem1)

    def fire_load(i, b):
        pltpu.async_copy(he3_hbm.at[wid, pl.ds(i * SCH, SCH)], rows[b], rsem[b])

    def wait_load(b):
        pltpu.make_async_copy(he3_hbm.at[0, pl.ds(0, SCH)], rows[b], rsem[b]).wait()

    fire_load(0, 0)
    fire_load(1, 1)

    def pair(k, carry):
        for b in range(2):
            i = 2 * k + b
            wait_load(b)
            pltpu.sync_copy(rows[b], shared.at[idxm.at[i]], add=True)

            @pl.when(i + 2 < SNCH)
            def _():
                fire_load(i + 2, b)
        return carry

    lax.fori_loop(0, SNCH // 2, pair, 0)
    plsc.subcore_barrier()
    pltpu.sync_copy(shared.at[pl.ds(s * RPT, RPT)], out_hbm.at[c, pl.ds(s * RPT, RPT)])


def _row(v):
    return v.reshape(1, H)


def kernel(h_node, h_edge, edge_index,
           e_W1, e_b1, e_g1, e_be1, e_W2, e_b2,
           n_W1, n_b1, n_g1, n_be1, n_W2, n_b2,
           en_g, en_b, nn_g, nn_b):
    src = edge_index[0].astype(jnp.int32)
    dst = edge_index[1].astype(jnp.int32)
    w1s, w1d, w1e = e_W1[:H], e_W1[H:2 * H], e_W1[2 * H:]
    nw1a, nw1b = n_W1[:H], n_W1[H:]

    BP = 2000
    ps, pd = pl.pallas_call(
        _proj_body,
        grid=(N // BP,),
        in_specs=[
            pl.BlockSpec((BP, H), lambda i: (i, 0)),
            pl.BlockSpec((H, H), lambda i: (0, 0)),
            pl.BlockSpec((H, H), lambda i: (0, 0)),
        ],
        out_specs=[
            pl.BlockSpec((BP, H), lambda i: (i, 0)),
            pl.BlockSpec((BP, H), lambda i: (i, 0)),
        ],
        out_shape=[jax.ShapeDtypeStruct((N, H), F32)] * 2,
    )(h_node, w1s, w1d)

    src3 = src.reshape(NW, NCHUNK, CH)
    dst3 = dst.reshape(NW, NCHUNK, CH)
    mesh = plsc.VectorSubcoreMesh(core_axis_name="c", subcore_axis_name="s")
    gs, gd = pl.kernel(
        _gather_body,
        out_type=[jax.ShapeDtypeStruct((E, H), F32)] * 2,
        mesh=mesh,
        scratch_types=(
            [pltpu.VMEM((NCHUNK, CH), jnp.int32)] * 2
            + [pltpu.VMEM((CH, H), F32)] * 8
            + [pltpu.SemaphoreType.DMA] * 8
        ),
    )(ps, pd, src3, dst3)

    BE = 10000
    wspec = pl.BlockSpec((H, H), lambda i: (0, 0))
    vspec = pl.BlockSpec((1, H), lambda i: (0, 0))
    bspec = pl.BlockSpec((BE, H), lambda i: (i, 0))
    h_edge_new = pl.pallas_call(
        _edge_body,
        grid=(E // BE,),
        in_specs=[bspec, bspec, bspec, wspec, vspec, vspec, vspec, wspec,
                  vspec, vspec, vspec],
        out_specs=bspec,
        out_shape=jax.ShapeDtypeStruct((E, H), F32),
    )(gs, gd, h_edge, w1e, _row(e_b1), _row(e_g1), _row(e_be1), e_W2,
      _row(e_b2), _row(en_g), _row(en_b))

    aggp = pl.kernel(
        _scatter_body,
        out_type=jax.ShapeDtypeStruct((NC, NP, H), F32),
        mesh=plsc.VectorSubcoreMesh(core_axis_name="c", subcore_axis_name="s"),
        scratch_types=(
            [pltpu.VMEM((SNCH, SCH), jnp.int32)]
            + [pltpu.VMEM((SCH, H), F32)] * 2
            + [pltpu.VMEM_SHARED((NP, H), F32)]
            + [pltpu.SemaphoreType.DMA] * 2
        ),
    )(h_edge_new.reshape(NW, EPW, H), dst.reshape(NW, SNCH, SCH),
      jnp.zeros((NP, H), F32))

    BN = 2000
    wspecn = pl.BlockSpec((H, H), lambda i: (0, 0))
    vspecn = pl.BlockSpec((1, H), lambda i: (0, 0))
    h_node_new = pl.pallas_call(
        _node_body,
        grid=(N // BN,),
        in_specs=[
            pl.BlockSpec((BN, H), lambda i: (i, 0)),
            pl.BlockSpec((NC, BN, H), lambda i: (0, i, 0)),
            wspecn, wspecn, vspecn, vspecn, vspecn, wspecn, vspecn,
            vspecn, vspecn,
        ],
        out_specs=pl.BlockSpec((BN, H), lambda i: (i, 0)),
        out_shape=jax.ShapeDtypeStruct((N, H), F32),
    )(h_node, aggp, nw1a, nw1b, _row(n_b1), _row(n_g1), _row(n_be1), n_W2,
      _row(n_b2), _row(nn_g), _row(nn_b))

    return h_node_new, h_edge_new
